# baseline (speedup 1.0000x reference)
import functools

import jax
import jax.numpy as jnp
from jax import lax
from jax.experimental import pallas as pl
from jax.experimental.pallas import tpu as pltpu
from jax.experimental.pallas import tpu_sc as plsc

NUM_CORES = 2
NUM_SUBCORES = 16
NUM_WORKERS = NUM_CORES * NUM_SUBCORES

CHUNK_ROWS = 16


def _pos_embed_body(batch, seq_len, d_model, chunks_per_worker,
                    w_hbm, sp_hbm, out_hbm, sp_v, buf_v, sem_in, sem_out):
    core = lax.axis_index("c")
    sub = lax.axis_index("s")
    wid = sub * NUM_CORES + core
    rows_per_worker = chunks_per_worker * CHUNK_ROWS
    base = wid * rows_per_worker

    pltpu.sync_copy(sp_hbm, sp_v)
    start = pl.multiple_of(sp_v[...][0], 8)

    for c in range(chunks_per_worker):
        row0 = base + c * CHUNK_ROWS
        pltpu.async_copy(
            w_hbm.at[pl.ds(start + row0, CHUNK_ROWS)], buf_v, sem_in
        ).wait()
        copies = []
        for b in range(batch):
            copies.append(pltpu.async_copy(
                buf_v, out_hbm.at[b, pl.ds(row0, CHUNK_ROWS)], sem_out))
        for cp in copies:
            cp.wait()


def kernel(tokens, start_pos, W_pos):
    batch, seq_len = tokens.shape
    d_model = W_pos.shape[-1]
    assert seq_len % (NUM_WORKERS * CHUNK_ROWS) == 0
    chunks_per_worker = seq_len // (NUM_WORKERS * CHUNK_ROWS)

    sp_arr = jnp.full((16,), start_pos, dtype=jnp.int32)

    mesh = plsc.VectorSubcoreMesh(
        core_axis_name="c", subcore_axis_name="s",
        num_cores=NUM_CORES, num_subcores=NUM_SUBCORES)

    body = functools.partial(
        _pos_embed_body, batch, seq_len, d_model, chunks_per_worker)

    out = pl.kernel(
        body,
        out_type=jax.ShapeDtypeStruct((batch, seq_len, d_model), W_pos.dtype),
        mesh=mesh,
        scratch_types=[
            pltpu.VMEM((16,), jnp.int32),
            pltpu.VMEM((CHUNK_ROWS, d_model), W_pos.dtype),
            pltpu.SemaphoreType.DMA,
            pltpu.SemaphoreType.DMA,
        ],
    )(W_pos, sp_arr)
    return out
